# Initial kernel scaffold; baseline (speedup 1.0000x reference)
import jax
import jax.numpy as jnp
from jax.experimental import pallas as pl


def kernel(input_ids, tok_emb, pos_emb):
    raise NotImplementedError("write your pallas kernel here")


# speedup vs baseline: 2.9360x; 2.9360x over previous
import functools

import jax
import jax.numpy as jnp
from jax import lax
from jax.experimental import pallas as pl
from jax.experimental.pallas import tpu as pltpu
from jax.experimental.pallas import tpu_sc as plsc

_BATCH = 4
_SEQ = 8192
_HIDDEN = 128
_LANES = 16
_NC = 2
_NS = 16
_NW = _NC * _NS
_SEQ_W = _SEQ // _NW
_CHUNK = 128
_NCHUNK = _SEQ_W // _CHUNK


def _body(ids_hbm, tok_hbm, pos_hbm, out_hbm, idx_v, pos_v, tok_v, sem):
    wid = lax.axis_index("s") * _NC + lax.axis_index("c")
    seq0 = wid * _SEQ_W

    pltpu.sync_copy(pos_hbm.at[pl.ds(seq0, _SEQ_W)], pos_v)

    for b in range(_BATCH):
        pltpu.sync_copy(ids_hbm.at[b, pl.ds(wid * _NCHUNK, _NCHUNK)], idx_v)
        copies = [
            pltpu.async_copy(
                tok_hbm.at[idx_v.at[j]],
                tok_v.at[pl.ds(j * _CHUNK, _CHUNK)],
                sem,
            )
            for j in range(_NCHUNK)
        ]
        for c in copies:
            c.wait()

        @plsc.parallel_loop(0, _SEQ_W, 1, unroll=4)
        def _add_row(r):
            for j in range(_HIDDEN // _LANES):
                x = pos_v[r, pl.ds(j * _LANES, _LANES)]
                plsc.addupdate(tok_v.at[r, pl.ds(j * _LANES, _LANES)], x)

        pltpu.sync_copy(tok_v, out_hbm.at[b, pl.ds(seq0, _SEQ_W)])


def kernel(input_ids, tok_emb, pos_emb):
    bsz, seq_len = input_ids.shape
    ids3 = input_ids.reshape(_BATCH, _SEQ // _CHUNK, _CHUNK).astype(jnp.int32)

    k = pl.kernel(
        _body,
        out_type=jax.ShapeDtypeStruct((_BATCH, _SEQ, _HIDDEN), jnp.float32),
        mesh=plsc.VectorSubcoreMesh(core_axis_name="c", subcore_axis_name="s"),
        scratch_types=[
            pltpu.VMEM((_NCHUNK, _CHUNK), jnp.int32),
            pltpu.VMEM((_SEQ_W, _HIDDEN), jnp.float32),
            pltpu.VMEM((_SEQ_W, _HIDDEN), jnp.float32),
            pltpu.SemaphoreType.DMA,
        ],
    )
    x = k(ids3, tok_emb, pos_emb)
    pos = jnp.broadcast_to(
        jnp.arange(seq_len, dtype=input_ids.dtype)[None, :], (bsz, seq_len)
    )
    return (x, pos)


# speedup vs baseline: 3.5003x; 1.1922x over previous
import jax
import jax.numpy as jnp
from jax import lax
from jax.experimental import pallas as pl
from jax.experimental.pallas import tpu as pltpu
from jax.experimental.pallas import tpu_sc as plsc

_BATCH = 4
_SEQ = 8192
_HIDDEN = 128
_LANES = 16
_NC = 2
_NS = 16
_NW = _NC * _NS
_SEQ_W = _SEQ // _NW
_CHUNK = 128
_NCHUNK = _SEQ_W // _CHUNK


def _body(
    ids_hbm, tok_hbm, pos_hbm, out_hbm,
    idx_v, tok0, tok1, pos_v,
    gsem0, gsem1, osem0, osem1,
):
    wid = lax.axis_index("s") * _NC + lax.axis_index("c")
    seq0 = wid * _SEQ_W
    bufs = [tok0, tok1]
    gsems = [gsem0, gsem1]
    osems = [osem0, osem1]

    pltpu.sync_copy(ids_hbm.at[wid], idx_v)
    pltpu.sync_copy(pos_hbm.at[pl.ds(seq0, _SEQ_W)], pos_v)

    def fire_gather(b, buf, sem):
        return [
            pltpu.async_copy(
                tok_hbm.at[idx_v.at[b, j]],
                buf.at[pl.ds(j * _CHUNK, _CHUNK)],
                sem,
            )
            for j in range(_NCHUNK)
        ]

    out_copies = [None, None]
    pending = fire_gather(0, bufs[0], gsems[0])
    for b in range(_BATCH):
        nxt = (b + 1) % 2
        if b + 1 < _BATCH:
            if out_copies[nxt] is not None:
                out_copies[nxt].wait()
            next_pending = fire_gather(b + 1, bufs[nxt], gsems[nxt])
        for c in pending:
            c.wait()

        buf = bufs[b % 2]

        @plsc.parallel_loop(0, _SEQ_W, 1, unroll=4)
        def _add_row(r):
            for j in range(_HIDDEN // _LANES):
                x = pos_v[r, pl.ds(j * _LANES, _LANES)]
                plsc.addupdate(buf.at[r, pl.ds(j * _LANES, _LANES)], x)

        out_copies[b % 2] = pltpu.async_copy(
            buf, out_hbm.at[b, pl.ds(seq0, _SEQ_W)], osems[b % 2]
        )
        if b + 1 < _BATCH:
            pending = next_pending

    for c in out_copies:
        c.wait()


def kernel(input_ids, tok_emb, pos_emb):
    bsz, seq_len = input_ids.shape
    ids_r = jnp.transpose(
        input_ids.astype(jnp.int32).reshape(_BATCH, _NW, _NCHUNK, _CHUNK),
        (1, 0, 2, 3),
    )

    k = pl.kernel(
        _body,
        out_type=jax.ShapeDtypeStruct((_BATCH, _SEQ, _HIDDEN), jnp.float32),
        mesh=plsc.VectorSubcoreMesh(core_axis_name="c", subcore_axis_name="s"),
        scratch_types=[
            pltpu.VMEM((_BATCH, _NCHUNK, _CHUNK), jnp.int32),
            pltpu.VMEM((_SEQ_W, _HIDDEN), jnp.float32),
            pltpu.VMEM((_SEQ_W, _HIDDEN), jnp.float32),
            pltpu.VMEM((_SEQ_W, _HIDDEN), jnp.float32),
            pltpu.SemaphoreType.DMA,
            pltpu.SemaphoreType.DMA,
            pltpu.SemaphoreType.DMA,
            pltpu.SemaphoreType.DMA,
        ],
    )
    x = k(ids_r, tok_emb, pos_emb)
    pos = jnp.broadcast_to(
        jnp.arange(seq_len, dtype=input_ids.dtype)[None, :], (bsz, seq_len)
    )
    return (x, pos)
